# baseline (speedup 1.0000x reference)
import functools

import jax
import jax.numpy as jnp
from jax import lax
from jax.experimental import pallas as pl
from jax.experimental.pallas import tpu as pltpu
from jax.experimental.pallas import tpu_sc as plsc


@functools.cache
def _make_lookup(BL: int, E: int, C: int):
    info = plsc.get_sparse_core_info()
    NC, NS = info.num_cores, info.num_subcores
    NW = NC * NS
    per_w = BL // NW
    n_chunks = per_w // C
    assert BL % NW == 0 and per_w % C == 0 and n_chunks % 2 == 0

    mesh = plsc.VectorSubcoreMesh(core_axis_name="c", subcore_axis_name="s")

    @functools.partial(
        pl.kernel,
        mesh=mesh,
        compiler_params=pltpu.CompilerParams(use_tc_tiling_on_sc=False),
        out_type=jax.ShapeDtypeStruct((2, BL, E), jnp.float32),
        scratch_types=[
            pltpu.VMEM((C,), jnp.int32),
            pltpu.VMEM((C,), jnp.int32),
            pltpu.VMEM((C, E), jnp.float32),
            pltpu.VMEM((C, E), jnp.float32),
            pltpu.SemaphoreType.DMA,
            pltpu.SemaphoreType.DMA,
            pltpu.SemaphoreType.DMA,
            pltpu.SemaphoreType.DMA,
        ],
    )
    def lookup(table, idx1, idx2, out, i0, i1, r0, r1, g0, g1, w0, w1):
        wid = lax.axis_index("s") * NC + lax.axis_index("c")
        base = wid * per_w

        def body(i, _):
            for s, idx in ((0, idx1), (1, idx2)):
                off0 = base + (2 * i) * C
                off1 = off0 + C
                pltpu.sync_copy(idx.at[pl.ds(off0, C)], i0)
                ga = pltpu.async_copy(table.at[i0], r0, g0)
                pltpu.sync_copy(idx.at[pl.ds(off1, C)], i1)
                gb = pltpu.async_copy(table.at[i1], r1, g1)
                ga.wait()
                wa = pltpu.async_copy(r0, out.at[s, pl.ds(off0, C)], w0)
                gb.wait()
                wb = pltpu.async_copy(r1, out.at[s, pl.ds(off1, C)], w1)
                wa.wait()
                wb.wait()
            return ()

        lax.fori_loop(0, n_chunks // 2, body, ())

    return lookup


@functools.cache
def _make_repack(B: int, L: int, E: int):
    info = plsc.get_sparse_core_info()
    NC, NS = info.num_cores, info.num_subcores
    NW = NC * NS
    pairs = 2 * B
    per_w = pairs // NW
    R = L // 4
    assert pairs % NW == 0 and per_w % 4 == 0 and L % 8 == 0

    mesh = plsc.VectorSubcoreMesh(core_axis_name="c", subcore_axis_name="s")

    @functools.partial(
        pl.kernel,
        mesh=mesh,
        compiler_params=pltpu.CompilerParams(use_tc_tiling_on_sc=True),
        out_type=jax.ShapeDtypeStruct((2, B, L, E), jnp.float32),
        scratch_types=[
            pltpu.VMEM((4 * R, 128), jnp.float32),
            pltpu.VMEM((L, E), jnp.float32),
            pltpu.SemaphoreType.DMA,
            pltpu.SemaphoreType.DMA,
        ],
    )
    def repack(x, out, vin, vout, isem, osem):
        wid = lax.axis_index("s") * NC + lax.axis_index("c")
        base = wid * per_w

        def group(g, _):
            p0 = base + 4 * g
            pltpu.async_copy(x.at[pl.ds(p0 * R, 4 * R)], vin, isem).wait()
            for pb in range(4):
                p = p0 + pb

                def row(r, _):
                    for k in range(4):
                        for e0 in range(0, E, 16):
                            vout[4 * r + k, pl.ds(e0, 16)] = vin[
                                pb * R + r, pl.ds(k * E + e0, 16)
                            ]
                    return ()

                lax.fori_loop(0, R, row, ())
                pltpu.async_copy(
                    vout, out.at[p // B, p % B], osem
                ).wait()
            return ()

        lax.fori_loop(0, per_w // 4, group, ())

    return repack


def kernel(embeddings, input1, input2):
    b, l, nf = input1.shape
    e = embeddings.shape[1]
    BL = b * l * nf
    idx1 = input1.reshape(BL)
    idx2 = input2.reshape(BL)
    out = _make_lookup(BL, e, 1600)(embeddings, idx1, idx2)
    out128 = jax.lax.optimization_barrier(out.reshape(2 * BL * e // 128, 128))
    return _make_repack(b, l * nf, e)(out128)


# speedup vs baseline: 1.1990x; 1.1990x over previous
import functools

import jax
import jax.numpy as jnp
from jax import lax
from jax.experimental import pallas as pl
from jax.experimental.pallas import tpu as pltpu
from jax.experimental.pallas import tpu_sc as plsc


@functools.cache
def _make_lookup(BL: int, E: int, C: int):
    info = plsc.get_sparse_core_info()
    NC, NS = info.num_cores, info.num_subcores
    NW = NC * NS
    per_w = BL // NW
    n_chunks = per_w // C
    assert BL % NW == 0 and per_w % C == 0 and n_chunks % 2 == 0

    mesh = plsc.VectorSubcoreMesh(core_axis_name="c", subcore_axis_name="s")

    @functools.partial(
        pl.kernel,
        mesh=mesh,
        compiler_params=pltpu.CompilerParams(use_tc_tiling_on_sc=False),
        out_type=jax.ShapeDtypeStruct((2, BL, E), jnp.float32),
        scratch_types=[
            pltpu.VMEM((C,), jnp.int32),
            pltpu.VMEM((C,), jnp.int32),
            pltpu.VMEM((C, E), jnp.float32),
            pltpu.VMEM((C, E), jnp.float32),
            pltpu.SemaphoreType.DMA,
            pltpu.SemaphoreType.DMA,
            pltpu.SemaphoreType.DMA,
            pltpu.SemaphoreType.DMA,
        ],
    )
    def lookup(table, idx1, idx2, out, i0, i1, r0, r1, g0, g1, w0, w1):
        wid = lax.axis_index("s") * NC + lax.axis_index("c")
        base = wid * per_w

        def body(i, _):
            for s, idx in ((0, idx1), (1, idx2)):
                off0 = base + (2 * i) * C
                off1 = off0 + C
                pltpu.sync_copy(idx.at[pl.ds(off0, C)], i0)
                ga = pltpu.async_copy(table.at[i0], r0, g0)
                pltpu.sync_copy(idx.at[pl.ds(off1, C)], i1)
                gb = pltpu.async_copy(table.at[i1], r1, g1)
                ga.wait()
                wa = pltpu.async_copy(r0, out.at[s, pl.ds(off0, C)], w0)
                gb.wait()
                wb = pltpu.async_copy(r1, out.at[s, pl.ds(off1, C)], w1)
                wa.wait()
                wb.wait()
            return ()

        lax.fori_loop(0, n_chunks // 2, body, ())

    return lookup


@functools.cache
def _make_repack(B: int, L: int, E: int):
    info = plsc.get_sparse_core_info()
    NC, NS = info.num_cores, info.num_subcores
    NW = NC * NS
    pairs = 2 * B
    per_w = pairs // NW
    R = L // 4
    assert pairs % NW == 0 and per_w % 4 == 0 and L % 8 == 0

    mesh = plsc.VectorSubcoreMesh(core_axis_name="c", subcore_axis_name="s")

    @functools.partial(
        pl.kernel,
        mesh=mesh,
        compiler_params=pltpu.CompilerParams(use_tc_tiling_on_sc=True),
        out_type=jax.ShapeDtypeStruct((2, B, L, E), jnp.float32),
        scratch_types=[
            pltpu.VMEM((4 * R, 128), jnp.float32),
            pltpu.VMEM((4 * R, 128), jnp.float32),
            pltpu.VMEM((L, E), jnp.float32),
            pltpu.VMEM((L, E), jnp.float32),
            pltpu.SemaphoreType.DMA,
            pltpu.SemaphoreType.DMA,
            pltpu.SemaphoreType.DMA,
            pltpu.SemaphoreType.DMA,
        ],
    )
    def repack(x, out, vinA, vinB, voutA, voutB, isA, isB, osA, osB):
        wid = lax.axis_index("s") * NC + lax.axis_index("c")
        base = wid * per_w
        n_groups = per_w // 4

        pltpu.make_async_copy(x.at[pl.ds(base * R, 4 * R)], vinA, isA).start()

        def do_group(g, vin, isem_next, vin_next, my_wait):
            p0 = base + 4 * g
            my_wait()
            nxt = p0 + 4
            nd = pltpu.make_async_copy(
                x.at[pl.ds(nxt * R, 4 * R)], vin_next, isem_next)

            @pl.when(g + 1 < n_groups)
            def _():
                nd.start()

            write_descs = []
            for pb in range(4):
                p = p0 + pb
                vout = (voutA, voutB)[pb % 2]
                if len(write_descs) >= 2:
                    write_descs[pb - 2].wait()

                def row(r, _, pb=pb, vout=vout):
                    for k in range(4):
                        for e0 in range(0, E, 16):
                            vout[4 * r + k, pl.ds(e0, 16)] = vin[
                                pb * R + r, pl.ds(k * E + e0, 16)
                            ]
                    return ()

                lax.fori_loop(0, R, row, (), unroll=5)
                d = pltpu.make_async_copy(
                    vout, out.at[p // B, p % B], (osA, osB)[pb % 2])
                d.start()
                write_descs.append(d)
            write_descs[2].wait()
            write_descs[3].wait()

        def body(i, _):
            gA = 2 * i
            gB = 2 * i + 1
            do_group(
                gA, vinA, isB, vinB,
                lambda: pltpu.make_async_copy(
                    x.at[pl.ds(0, 4 * R)], vinA, isA).wait(),
            )
            do_group(
                gB, vinB, isA, vinA,
                lambda: pltpu.make_async_copy(
                    x.at[pl.ds(0, 4 * R)], vinB, isB).wait(),
            )
            return ()

        lax.fori_loop(0, n_groups // 2, body, ())

    return repack


def kernel(embeddings, input1, input2):
    b, l, nf = input1.shape
    e = embeddings.shape[1]
    BL = b * l * nf
    idx1 = input1.reshape(BL)
    idx2 = input2.reshape(BL)
    out = _make_lookup(BL, e, 1600)(embeddings, idx1, idx2)
    out128 = jax.lax.optimization_barrier(out.reshape(2 * BL * e // 128, 128))
    return _make_repack(b, l * nf, e)(out128)


# speedup vs baseline: 1.2965x; 1.0812x over previous
import functools

import jax
import jax.numpy as jnp
from jax import lax
from jax.experimental import pallas as pl
from jax.experimental.pallas import tpu as pltpu
from jax.experimental.pallas import tpu_sc as plsc


@functools.cache
def _make_lookup(BL: int, E: int, C: int):
    info = plsc.get_sparse_core_info()
    NC, NS = info.num_cores, info.num_subcores
    NW = NC * NS
    per_w = BL // NW
    n_chunks = per_w // C
    assert BL % NW == 0 and per_w % C == 0 and n_chunks % 2 == 0

    mesh = plsc.VectorSubcoreMesh(core_axis_name="c", subcore_axis_name="s")

    @functools.partial(
        pl.kernel,
        mesh=mesh,
        compiler_params=pltpu.CompilerParams(use_tc_tiling_on_sc=False),
        out_type=jax.ShapeDtypeStruct((2, BL, E), jnp.float32),
        scratch_types=[
            pltpu.VMEM((C,), jnp.int32),
            pltpu.VMEM((C,), jnp.int32),
            pltpu.VMEM((C, E), jnp.float32),
            pltpu.VMEM((C, E), jnp.float32),
            pltpu.SemaphoreType.DMA,
            pltpu.SemaphoreType.DMA,
            pltpu.SemaphoreType.DMA,
            pltpu.SemaphoreType.DMA,
        ],
    )
    def lookup(table, idx1, idx2, out, i0, i1, r0, r1, g0, g1, w0, w1):
        wid = lax.axis_index("s") * NC + lax.axis_index("c")
        base = wid * per_w

        def body(i, _):
            for s, idx in ((0, idx1), (1, idx2)):
                off0 = base + (2 * i) * C
                off1 = off0 + C
                pltpu.sync_copy(idx.at[pl.ds(off0, C)], i0)
                ga = pltpu.async_copy(table.at[i0], r0, g0)
                pltpu.sync_copy(idx.at[pl.ds(off1, C)], i1)
                gb = pltpu.async_copy(table.at[i1], r1, g1)
                ga.wait()
                wa = pltpu.async_copy(r0, out.at[s, pl.ds(off0, C)], w0)
                gb.wait()
                wb = pltpu.async_copy(r1, out.at[s, pl.ds(off1, C)], w1)
                wa.wait()
                wb.wait()
            return ()

        lax.fori_loop(0, n_chunks // 2, body, ())

    return lookup


def kernel(embeddings, input1, input2):
    b, l, nf = input1.shape
    e = embeddings.shape[1]
    BL = b * l * nf
    idx1 = input1.reshape(BL)
    idx2 = input2.reshape(BL)
    out = _make_lookup(BL, e, 1600)(embeddings, idx1, idx2)
    out128 = jax.lax.optimization_barrier(out.reshape(2 * BL * e // 128, 128))
    return out128.reshape(2, b, l, nf * e)
